# baseline (speedup 1.0000x reference)
import functools

import jax
import jax.numpy as jnp
from jax import lax
from jax.experimental import pallas as pl
from jax.experimental.pallas import tpu as pltpu
from jax.experimental.pallas import tpu_sc as plsc

_NC = 2
_NS = 16
_NW = _NC * _NS
_L = 16
_CH = 80
_SUB = 25
_EPS = 0.5


def _make_hist(N, E):
  cpt = E // _CH // _NW
  mesh = plsc.VectorSubcoreMesh(core_axis_name="c", subcore_axis_name="s")

  @functools.partial(
      pl.kernel,
      out_type=jax.ShapeDtypeStruct((_NC, N), jnp.float32),
      mesh=mesh,
      scratch_types=[
          pltpu.VMEM_SHARED((N,), jnp.float32),
          pltpu.VMEM((cpt, _CH), jnp.int32),
          pltpu.VMEM((_CH,), jnp.float32),
          pltpu.VMEM((2000,), jnp.float32),
          pltpu.SemaphoreType.DMA,
      ],
  )
  def hist(row_hbm, out_hbm, acc, rowi, ones_v, zbuf, sem):
    c = lax.axis_index("c")
    s = lax.axis_index("s")
    wid = s * _NC + c
    zv = jnp.zeros((_L,), jnp.float32)
    ov = jnp.ones((_L,), jnp.float32)
    for i in range(_CH // _L):
      ones_v[pl.ds(i * _L, _L)] = ov

    def zb(i, _):
      zbuf[pl.ds(i * _L, _L)] = zv
      return 0

    lax.fori_loop(0, 2000 // _L, zb, 0)

    @pl.when(s == 0)
    def _():
      for k in range(N // 2000):
        pltpu.sync_copy(zbuf, acc.at[pl.ds(k * 2000, 2000)])

    plsc.subcore_barrier()
    pltpu.sync_copy(row_hbm.at[wid], rowi)

    def fire(j):
      pltpu.async_copy(ones_v, acc.at[rowi.at[j]], sem, add=True)

    def drain(j):
      pltpu.make_async_copy(ones_v, acc.at[rowi.at[j]], sem).wait()

    lag = 8

    def head(j, _):
      fire(j)
      return 0

    def steady(j, _):
      fire(j + lag)
      drain(j)
      return 0

    def tailw(j, _):
      drain(j)
      return 0

    lax.fori_loop(0, lag, head, 0)
    lax.fori_loop(0, cpt - lag, steady, 0)
    lax.fori_loop(cpt - lag, cpt, tailw, 0)
    plsc.subcore_barrier()

    @pl.when(s == 0)
    def _():
      pltpu.sync_copy(acc, out_hbm.at[c])

  return hist


def _make_prop(N, H, E):
  cpt = E // _CH // _NW
  stg = cpt // _SUB
  spn = N // _NS // 8 * 8
  tail = N - spn * _NS
  mesh = plsc.VectorSubcoreMesh(core_axis_name="c", subcore_axis_name="s")

  @functools.partial(
      pl.kernel,
      out_type=jax.ShapeDtypeStruct((_NC, N, H), jnp.float32),
      mesh=mesh,
      scratch_types=[
          pltpu.VMEM_SHARED((N, H), jnp.float32),
          pltpu.VMEM((_SUB, _CH), jnp.int32),
          pltpu.VMEM((_SUB, _CH), jnp.int32),
          pltpu.VMEM((_CH, H), jnp.float32),
          pltpu.VMEM((_CH, H), jnp.float32),
          pltpu.VMEM((_CH, H), jnp.float32),
          pltpu.VMEM((_SUB, _CH), jnp.float32),
          pltpu.VMEM((_SUB, _CH), jnp.float32),
          pltpu.VMEM((16, H), jnp.float32),
          pltpu.SemaphoreType.DMA,
          pltpu.SemaphoreType.DMA,
          pltpu.SemaphoreType.DMA,
          pltpu.SemaphoreType.DMA,
          pltpu.SemaphoreType.DMA,
          pltpu.SemaphoreType.DMA,
          pltpu.SemaphoreType.DMA,
          pltpu.SemaphoreType.DMA,
          pltpu.SemaphoreType.DMA,
      ],
  )
  def prop(hs_hbm, a_hbm, b_hbm, row_hbm, col_hbm, out_hbm,
           acc, rowi, coli, rows0, rows1, rows2, va, vb, zbuf,
           semg0, semg1, semg2, semc0, semc1, semc2,
           semab0, semab1, semab2):
    c = lax.axis_index("c")
    s = lax.axis_index("s")
    wid = s * _NC + c
    zv = jnp.zeros((_L,), jnp.float32)

    def zb(i, _):
      r = zbuf.at[i]
      for f in range(H // _L):
        r[pl.ds(f * _L, _L)] = zv
      return 0

    lax.fori_loop(0, 16, zb, 0)

    def zc(k, _):
      pltpu.async_copy(zbuf, acc.at[pl.ds(s * spn + k * 16, 16)], semc0)
      return 0

    def zw(k, _):
      pltpu.make_async_copy(zbuf, acc.at[pl.ds(s * spn + k * 16, 16)],
                            semc0).wait()
      return 0

    lax.fori_loop(0, spn // 16, zc, 0)
    lax.fori_loop(0, spn // 16, zw, 0)

    @pl.when(s == _NS - 1)
    def _():
      pltpu.sync_copy(zbuf.at[pl.ds(0, tail)], acc.at[pl.ds(spn * _NS, tail)])

    plsc.subcore_barrier()

    def scale(cur, k):
      arow = va.at[k]
      brow = vb.at[k]

      def grp(gi, _):
        sl = pl.ds(gi * _L, _L)
        sv = arow[sl] + brow[sl]
        x = jnp.minimum(jnp.abs(sv) * 2.0, 60.0)
        ex = jnp.exp(x)
        th = 1.0 - 2.0 / (ex + 1.0)
        nv = jnp.where(sv < 0.0, -th, th)
        for e in range(_L):
          nb = jnp.full((_L,), nv[e], jnp.float32)
          rr = cur.at[gi * _L + e]
          for f in range(H // _L):
            slf = pl.ds(f * _L, _L)
            rr[slf] = rr[slf] * nb
        return 0

      lax.fori_loop(0, _CH // _L, grp, 0)

    def wait_gather(k, cur, sem):
      pltpu.make_async_copy(hs_hbm.at[rowi.at[k]], cur, sem).wait()

    def stage(g, _):
      pltpu.sync_copy(row_hbm.at[wid, g], rowi)
      pltpu.sync_copy(col_hbm.at[wid, g], coli)

      bufs = (rows0, rows1, rows2)
      sgs = (semg0, semg1, semg2)
      scs = (semc0, semc1, semc2)
      sabs = (semab0, semab1, semab2)

      def gath(k, bi):
        pltpu.async_copy(hs_hbm.at[rowi.at[k]], bufs[bi], sgs[bi])

      def fire_ab(k, bi):
        pltpu.async_copy(a_hbm.at[rowi.at[k]], va.at[k], sabs[bi])
        pltpu.async_copy(b_hbm.at[coli.at[k]], vb.at[k], sabs[bi])

      def drain_ab(k, bi):
        pltpu.make_async_copy(a_hbm.at[rowi.at[k]], va.at[k], sabs[bi]).wait()
        pltpu.make_async_copy(b_hbm.at[coli.at[k]], vb.at[k], sabs[bi]).wait()

      def wait_scatter(k, bi):
        pltpu.make_async_copy(bufs[bi], acc.at[coli.at[k]], scs[bi]).wait()

      def proc(k, bi, wait_prev, pref):
        wait_gather(k, bufs[bi], sgs[bi])
        if wait_prev:
          wait_scatter(k - 1, (bi + 2) % 3)
        if pref:
          gath(k + 2, (bi + 2) % 3)
          fire_ab(k + 2, (bi + 2) % 3)
        drain_ab(k, bi)
        scale(bufs[bi], k)
        pltpu.async_copy(bufs[bi], acc.at[coli.at[k]], scs[bi], add=True)

      gath(0, 0)
      fire_ab(0, 0)
      gath(1, 1)
      fire_ab(1, 1)
      proc(0, 0, False, True)
      proc(1, 1, True, True)

      def three(m, _):
        k = 3 * m + 2
        proc(k, 2, True, True)
        proc(k + 1, 0, True, True)
        proc(k + 2, 1, True, True)
        return 0

      lax.fori_loop(0, (_SUB - 4) // 3, three, 0)
      proc(_SUB - 2, 2, True, False)
      proc(_SUB - 1, 0, True, False)
      wait_scatter(_SUB - 1, 0)
      return 0

    lax.fori_loop(0, stg, stage, 0)
    plsc.subcore_barrier()
    pltpu.sync_copy(acc.at[pl.ds(s * spn, spn)],
                    out_hbm.at[c, pl.ds(s * spn, spn)])

    @pl.when(s == _NS - 1)
    def _():
      pltpu.sync_copy(acc.at[pl.ds(spn * _NS, tail)],
                      out_hbm.at[c, pl.ds(spn * _NS, tail)])

  return prop


def _prep_tc(h, t1_W, t1_b, G0, gb0, degp):
  N, D = h.shape
  H = t1_W.shape[0]

  def body(h_ref, w_ref, b_ref, g_ref, gb_ref, dg_ref,
           h1_ref, hs_ref, a_ref, bb_ref, dr_ref):
    x = lax.dot_general(h_ref[...], w_ref[...], (((1,), (1,)), ((), ())),
                        preferred_element_type=jnp.float32)
    h1 = jnp.maximum(x + b_ref[...], 0.0)
    ab = lax.dot_general(h1, g_ref[...], (((1,), (0,)), ((), ())),
                         preferred_element_type=jnp.float32)
    deg = dg_ref[0, :] + dg_ref[1, :]
    dr = lax.rsqrt(jnp.maximum(deg, 1.0))
    h1_ref[...] = h1
    hs_ref[...] = h1 * dr[:, None]
    a_ref[...] = ab[:, 0] + gb_ref[0, 0]
    bb_ref[...] = ab[:, 1]
    dr_ref[...] = dr

  return pl.pallas_call(
      body,
      out_shape=[
          jax.ShapeDtypeStruct((N, H), jnp.float32),
          jax.ShapeDtypeStruct((N, H), jnp.float32),
          jax.ShapeDtypeStruct((N,), jnp.float32),
          jax.ShapeDtypeStruct((N,), jnp.float32),
          jax.ShapeDtypeStruct((N,), jnp.float32),
      ],
  )(h, t1_W, t1_b, G0, gb0, degp)


def _mid_tc(h1, p, G1, gb1, dr):
  N, H = h1.shape

  def body(h1_ref, p_ref, g_ref, gb_ref, dr_ref, hs_ref, a_ref, bb_ref):
    dr_col = dr_ref[...][:, None]
    h2 = _EPS * h1_ref[...] + dr_col * (p_ref[0] + p_ref[1])
    ab = lax.dot_general(h2, g_ref[...], (((1,), (0,)), ((), ())),
                         preferred_element_type=jnp.float32)
    hs_ref[...] = h2 * dr_col
    a_ref[...] = ab[:, 0] + gb_ref[0, 0]
    bb_ref[...] = ab[:, 1]

  return pl.pallas_call(
      body,
      out_shape=[
          jax.ShapeDtypeStruct((N, H), jnp.float32),
          jax.ShapeDtypeStruct((N,), jnp.float32),
          jax.ShapeDtypeStruct((N,), jnp.float32),
      ],
  )(h1, p, G1, gb1, dr)


def _fin_tc(h1, p, dr, t2_W, t2_b):
  N, H = h1.shape
  C = t2_W.shape[0]

  def body(h1_ref, p_ref, dr_ref, w_ref, b_ref, o_ref):
    h3 = _EPS * h1_ref[...] + dr_ref[...][:, None] * (p_ref[0] + p_ref[1])
    lg = lax.dot_general(h3, w_ref[...], (((1,), (1,)), ((), ())),
                         preferred_element_type=jnp.float32) + b_ref[...]
    m = jnp.max(lg, axis=1, keepdims=True)
    lse = m + jnp.log(jnp.sum(jnp.exp(lg - m), axis=1, keepdims=True))
    o_ref[...] = lg - lse

  return pl.pallas_call(
      body,
      out_shape=jax.ShapeDtypeStruct((N, C), jnp.float32),
  )(h1, p, dr, t2_W, t2_b)


def kernel(h, edge_index, labels, t1_W, t1_b, t2_W, t2_b,
           gate_W0, gate_b0, gate_W1, gate_b1):
  N, D = h.shape
  H = t1_W.shape[0]
  E = edge_index.shape[1]
  C = t2_W.shape[0]

  cpt = E // _CH // _NW
  row2d = edge_index[0].reshape(_NW, cpt // _SUB, _SUB, _CH)
  col2d = edge_index[1].reshape(_NW, cpt // _SUB, _SUB, _CH)
  zpad = jnp.zeros((H, 6), jnp.float32)
  G0 = jnp.concatenate(
      [gate_W0[0, :H, None], gate_W0[0, H:, None], zpad], axis=1)
  G1 = jnp.concatenate(
      [gate_W1[0, :H, None], gate_W1[0, H:, None], zpad], axis=1)
  gb0 = jnp.broadcast_to(gate_b0.reshape(1, 1), (1, 8))
  gb1 = jnp.broadcast_to(gate_b1.reshape(1, 1), (1, 8))

  degp = _make_hist(N, E)(edge_index[0].reshape(_NW, cpt, _CH))
  h1, hs1, a0, b0, dr = _prep_tc(h, t1_W, t1_b.reshape(1, H), G0, gb0, degp)
  prop = _make_prop(N, H, E)
  p0 = prop(hs1, a0, b0, row2d, col2d)
  hs2, a1, b1 = _mid_tc(h1, p0, G1, gb1, dr)
  p1 = prop(hs2, a1, b1, row2d, col2d)
  return _fin_tc(h1, p1, dr, t2_W, t2_b.reshape(1, C))


# speedup vs baseline: 1.0160x; 1.0160x over previous
import functools

import jax
import jax.numpy as jnp
from jax import lax
from jax.experimental import pallas as pl
from jax.experimental.pallas import tpu as pltpu
from jax.experimental.pallas import tpu_sc as plsc

_NC = 2
_NS = 16
_NW = _NC * _NS
_L = 16
_CH = 80
_SUB = 25
_EPS = 0.5


def _make_hist(N, E):
  cpt = E // _CH // _NW
  mesh = plsc.VectorSubcoreMesh(core_axis_name="c", subcore_axis_name="s")

  @functools.partial(
      pl.kernel,
      out_type=jax.ShapeDtypeStruct((_NC, N), jnp.float32),
      mesh=mesh,
      scratch_types=[
          pltpu.VMEM_SHARED((N,), jnp.float32),
          pltpu.VMEM((cpt, _CH), jnp.int32),
          pltpu.VMEM((_CH,), jnp.float32),
          pltpu.VMEM((2000,), jnp.float32),
          pltpu.SemaphoreType.DMA,
      ],
  )
  def hist(row_hbm, out_hbm, acc, rowi, ones_v, zbuf, sem):
    c = lax.axis_index("c")
    s = lax.axis_index("s")
    wid = s * _NC + c
    zv = jnp.zeros((_L,), jnp.float32)
    ov = jnp.ones((_L,), jnp.float32)
    for i in range(_CH // _L):
      ones_v[pl.ds(i * _L, _L)] = ov

    def zb(i, _):
      zbuf[pl.ds(i * _L, _L)] = zv
      return 0

    lax.fori_loop(0, 2000 // _L, zb, 0)

    @pl.when(s == 0)
    def _():
      for k in range(N // 2000):
        pltpu.sync_copy(zbuf, acc.at[pl.ds(k * 2000, 2000)])

    plsc.subcore_barrier()
    pltpu.sync_copy(row_hbm.at[wid], rowi)

    def fire(j):
      pltpu.async_copy(ones_v, acc.at[rowi.at[j]], sem, add=True)

    def drain(j):
      pltpu.make_async_copy(ones_v, acc.at[rowi.at[j]], sem).wait()

    lag = 8

    def head(j, _):
      fire(j)
      return 0

    def steady(j, _):
      fire(j + lag)
      drain(j)
      return 0

    def tailw(j, _):
      drain(j)
      return 0

    lax.fori_loop(0, lag, head, 0)
    lax.fori_loop(0, cpt - lag, steady, 0)
    lax.fori_loop(cpt - lag, cpt, tailw, 0)
    plsc.subcore_barrier()

    @pl.when(s == 0)
    def _():
      pltpu.sync_copy(acc, out_hbm.at[c])

  return hist


def _make_prop(N, H, E):
  cpt = E // _CH // _NW
  stg = cpt // _SUB
  spn = N // _NS // 8 * 8
  tail = N - spn * _NS
  mesh = plsc.VectorSubcoreMesh(core_axis_name="c", subcore_axis_name="s")

  @functools.partial(
      pl.kernel,
      out_type=jax.ShapeDtypeStruct((_NC, N, H), jnp.float32),
      mesh=mesh,
      scratch_types=[
          pltpu.VMEM_SHARED((N, H), jnp.float32),
          pltpu.VMEM((_SUB, _CH), jnp.int32),
          pltpu.VMEM((_SUB, _CH), jnp.int32),
          pltpu.VMEM((_CH, H), jnp.float32),
          pltpu.VMEM((_CH, H), jnp.float32),
          pltpu.VMEM((_CH, H), jnp.float32),
          pltpu.VMEM((_SUB, _CH), jnp.float32),
          pltpu.VMEM((_SUB, _CH), jnp.float32),
          pltpu.VMEM((16, H), jnp.float32),
          pltpu.SemaphoreType.DMA,
          pltpu.SemaphoreType.DMA,
          pltpu.SemaphoreType.DMA,
          pltpu.SemaphoreType.DMA,
          pltpu.SemaphoreType.DMA,
          pltpu.SemaphoreType.DMA,
          pltpu.SemaphoreType.DMA,
          pltpu.SemaphoreType.DMA,
          pltpu.SemaphoreType.DMA,
      ],
  )
  def prop(hs_hbm, a_hbm, b_hbm, row_hbm, col_hbm, out_hbm,
           acc, rowi, coli, rows0, rows1, rows2, va, vb, zbuf,
           semg0, semg1, semg2, semc0, semc1, semc2,
           semab0, semab1, semab2):
    c = lax.axis_index("c")
    s = lax.axis_index("s")
    wid = s * _NC + c
    zv = jnp.zeros((_L,), jnp.float32)

    def zb(i, _):
      r = zbuf.at[i]
      for f in range(H // _L):
        r[pl.ds(f * _L, _L)] = zv
      return 0

    lax.fori_loop(0, 16, zb, 0)

    def zc(k, _):
      pltpu.async_copy(zbuf, acc.at[pl.ds(s * spn + k * 16, 16)], semc0)
      return 0

    def zw(k, _):
      pltpu.make_async_copy(zbuf, acc.at[pl.ds(s * spn + k * 16, 16)],
                            semc0).wait()
      return 0

    lax.fori_loop(0, spn // 16, zc, 0)
    lax.fori_loop(0, spn // 16, zw, 0)

    @pl.when(s == _NS - 1)
    def _():
      pltpu.sync_copy(zbuf.at[pl.ds(0, tail)], acc.at[pl.ds(spn * _NS, tail)])

    plsc.subcore_barrier()

    def scale(cur, k):
      arow = va.at[k]
      brow = vb.at[k]

      def grp(gi, _):
        sl = pl.ds(gi * _L, _L)
        sv = arow[sl] + brow[sl]
        x = jnp.minimum(jnp.abs(sv) * 2.0, 60.0)
        ex = jnp.exp(x)
        th = 1.0 - 2.0 / (ex + 1.0)
        nv = jnp.where(sv < 0.0, -th, th)
        for e in range(_L):
          nb = jnp.full((_L,), nv[e], jnp.float32)
          rr = cur.at[gi * _L + e]
          for f in range(H // _L):
            slf = pl.ds(f * _L, _L)
            rr[slf] = rr[slf] * nb
        return 0

      lax.fori_loop(0, _CH // _L, grp, 0)

    def wait_gather(k, cur, sem):
      pltpu.make_async_copy(hs_hbm.at[rowi.at[k]], cur, sem).wait()

    def stage(g, _):
      pltpu.sync_copy(row_hbm.at[wid, g], rowi)
      pltpu.sync_copy(col_hbm.at[wid, g], coli)

      bufs = (rows0, rows1, rows2)
      sgs = (semg0, semg1, semg2)
      scs = (semc0, semc1, semc2)
      sabs = (semab0, semab1, semab2)

      def gath(k, bi):
        pltpu.async_copy(hs_hbm.at[rowi.at[k]], bufs[bi], sgs[bi])

      def fire_ab(k, bi):
        pltpu.async_copy(a_hbm.at[rowi.at[k]], va.at[k], sabs[bi])
        pltpu.async_copy(b_hbm.at[coli.at[k]], vb.at[k], sabs[bi])

      def drain_ab(k, bi):
        pltpu.make_async_copy(a_hbm.at[rowi.at[k]], va.at[k], sabs[bi]).wait()
        pltpu.make_async_copy(b_hbm.at[coli.at[k]], vb.at[k], sabs[bi]).wait()

      def wait_scatter(k, bi):
        pltpu.make_async_copy(bufs[bi], acc.at[coli.at[k]], scs[bi]).wait()

      def proc(k, bi, wait_prev, pref):
        wait_gather(k, bufs[bi], sgs[bi])
        drain_ab(k, bi)
        scale(bufs[bi], k)
        pltpu.async_copy(bufs[bi], acc.at[coli.at[k]], scs[bi], add=True)
        if wait_prev:
          wait_scatter(k - 1, (bi + 2) % 3)
        if pref:
          gath(k + 2, (bi + 2) % 3)
          fire_ab(k + 2, (bi + 2) % 3)

      gath(0, 0)
      fire_ab(0, 0)
      gath(1, 1)
      fire_ab(1, 1)
      proc(0, 0, False, True)
      proc(1, 1, True, True)

      def three(m, _):
        k = 3 * m + 2
        proc(k, 2, True, True)
        proc(k + 1, 0, True, True)
        proc(k + 2, 1, True, True)
        return 0

      lax.fori_loop(0, (_SUB - 4) // 3, three, 0)
      proc(_SUB - 2, 2, True, False)
      proc(_SUB - 1, 0, True, False)
      wait_scatter(_SUB - 1, 0)
      return 0

    lax.fori_loop(0, stg, stage, 0)
    plsc.subcore_barrier()
    pltpu.sync_copy(acc.at[pl.ds(s * spn, spn)],
                    out_hbm.at[c, pl.ds(s * spn, spn)])

    @pl.when(s == _NS - 1)
    def _():
      pltpu.sync_copy(acc.at[pl.ds(spn * _NS, tail)],
                      out_hbm.at[c, pl.ds(spn * _NS, tail)])

  return prop


def _prep_tc(h, t1_W, t1_b, G0, gb0, degp):
  N, D = h.shape
  H = t1_W.shape[0]

  def body(h_ref, w_ref, b_ref, g_ref, gb_ref, dg_ref,
           h1_ref, hs_ref, a_ref, bb_ref, dr_ref):
    x = lax.dot_general(h_ref[...], w_ref[...], (((1,), (1,)), ((), ())),
                        preferred_element_type=jnp.float32)
    h1 = jnp.maximum(x + b_ref[...], 0.0)
    ab = lax.dot_general(h1, g_ref[...], (((1,), (0,)), ((), ())),
                         preferred_element_type=jnp.float32)
    deg = dg_ref[0, :] + dg_ref[1, :]
    dr = lax.rsqrt(jnp.maximum(deg, 1.0))
    h1_ref[...] = h1
    hs_ref[...] = h1 * dr[:, None]
    a_ref[...] = ab[:, 0] + gb_ref[0, 0]
    bb_ref[...] = ab[:, 1]
    dr_ref[...] = dr

  return pl.pallas_call(
      body,
      out_shape=[
          jax.ShapeDtypeStruct((N, H), jnp.float32),
          jax.ShapeDtypeStruct((N, H), jnp.float32),
          jax.ShapeDtypeStruct((N,), jnp.float32),
          jax.ShapeDtypeStruct((N,), jnp.float32),
          jax.ShapeDtypeStruct((N,), jnp.float32),
      ],
  )(h, t1_W, t1_b, G0, gb0, degp)


def _mid_tc(h1, p, G1, gb1, dr):
  N, H = h1.shape

  def body(h1_ref, p_ref, g_ref, gb_ref, dr_ref, hs_ref, a_ref, bb_ref):
    dr_col = dr_ref[...][:, None]
    h2 = _EPS * h1_ref[...] + dr_col * (p_ref[0] + p_ref[1])
    ab = lax.dot_general(h2, g_ref[...], (((1,), (0,)), ((), ())),
                         preferred_element_type=jnp.float32)
    hs_ref[...] = h2 * dr_col
    a_ref[...] = ab[:, 0] + gb_ref[0, 0]
    bb_ref[...] = ab[:, 1]

  return pl.pallas_call(
      body,
      out_shape=[
          jax.ShapeDtypeStruct((N, H), jnp.float32),
          jax.ShapeDtypeStruct((N,), jnp.float32),
          jax.ShapeDtypeStruct((N,), jnp.float32),
      ],
  )(h1, p, G1, gb1, dr)


def _fin_tc(h1, p, dr, t2_W, t2_b):
  N, H = h1.shape
  C = t2_W.shape[0]

  def body(h1_ref, p_ref, dr_ref, w_ref, b_ref, o_ref):
    h3 = _EPS * h1_ref[...] + dr_ref[...][:, None] * (p_ref[0] + p_ref[1])
    lg = lax.dot_general(h3, w_ref[...], (((1,), (1,)), ((), ())),
                         preferred_element_type=jnp.float32) + b_ref[...]
    m = jnp.max(lg, axis=1, keepdims=True)
    lse = m + jnp.log(jnp.sum(jnp.exp(lg - m), axis=1, keepdims=True))
    o_ref[...] = lg - lse

  return pl.pallas_call(
      body,
      out_shape=jax.ShapeDtypeStruct((N, C), jnp.float32),
  )(h1, p, dr, t2_W, t2_b)


def kernel(h, edge_index, labels, t1_W, t1_b, t2_W, t2_b,
           gate_W0, gate_b0, gate_W1, gate_b1):
  N, D = h.shape
  H = t1_W.shape[0]
  E = edge_index.shape[1]
  C = t2_W.shape[0]

  cpt = E // _CH // _NW
  row2d = edge_index[0].reshape(_NW, cpt // _SUB, _SUB, _CH)
  col2d = edge_index[1].reshape(_NW, cpt // _SUB, _SUB, _CH)
  zpad = jnp.zeros((H, 6), jnp.float32)
  G0 = jnp.concatenate(
      [gate_W0[0, :H, None], gate_W0[0, H:, None], zpad], axis=1)
  G1 = jnp.concatenate(
      [gate_W1[0, :H, None], gate_W1[0, H:, None], zpad], axis=1)
  gb0 = jnp.broadcast_to(gate_b0.reshape(1, 1), (1, 8))
  gb1 = jnp.broadcast_to(gate_b1.reshape(1, 1), (1, 8))

  degp = _make_hist(N, E)(edge_index[0].reshape(_NW, cpt, _CH))
  h1, hs1, a0, b0, dr = _prep_tc(h, t1_W, t1_b.reshape(1, H), G0, gb0, degp)
  prop = _make_prop(N, H, E)
  p0 = prop(hs1, a0, b0, row2d, col2d)
  hs2, a1, b1 = _mid_tc(h1, p0, G1, gb1, dr)
  p1 = prop(hs2, a1, b1, row2d, col2d)
  return _fin_tc(h1, p1, dr, t2_W, t2_b.reshape(1, C))
